# baseline (speedup 1.0000x reference)
import functools

import jax
import jax.numpy as jnp
from jax import lax
from jax.experimental import pallas as pl
from jax.experimental.pallas import tpu as pltpu
from jax.experimental.pallas import tpu_sc as plsc

N_LEVELS = 8
K = 1024
D = 256
N = 16384
BLOCK_B = 2048
N_SPLIT = 4


def _tc_level_kernel(r_ref, sel_ref, cbm2_ref, cnorm_ref, idx_ref, rout_ref,
                     *, subtract, write_r):
    b = r_ref.shape[0]
    h = b // N_SPLIT
    lane_iota = jax.lax.broadcasted_iota(jnp.int32, (h, K), 1)

    def chain(j):
        sl = pl.ds(j * h, h)
        r = r_ref[sl, :]
        if subtract:
            r = r - sel_ref[sl, :]
        if write_r:
            rout_ref[sl, :] = r
        m2p = jax.lax.dot_general(
            r, cbm2_ref[0], (((1,), (1,)), ((), ())),
            preferred_element_type=jnp.float32,
        )
        d2 = cnorm_ref[0][None, :] + m2p
        idx_ref[sl] = jnp.argmin(d2, axis=1).astype(jnp.int32)

    for j in range(N_SPLIT):
        chain(j)


def _tc_level(r, sel, cbm2_l, cn_l, subtract, write_r):
    grid = (N // BLOCK_B,)
    body = functools.partial(_tc_level_kernel, subtract=subtract,
                             write_r=write_r)
    in_specs = [
        pl.BlockSpec((BLOCK_B, D), lambda i: (i, 0)),
        pl.BlockSpec((BLOCK_B, D), lambda i: (i, 0)),
        pl.BlockSpec((1, K, D), lambda i: (0, 0, 0)),
        pl.BlockSpec((1, K), lambda i: (0, 0)),
    ]
    out_specs = [
        pl.BlockSpec((BLOCK_B,), lambda i: (i,)),
        pl.BlockSpec((BLOCK_B, D), lambda i: (i, 0)),
    ]
    out_shape = [
        jax.ShapeDtypeStruct((N,), jnp.int32),
        jax.ShapeDtypeStruct((N, D), jnp.float32),
    ]
    idx, rout = pl.pallas_call(
        body,
        grid=grid,
        in_specs=in_specs,
        out_specs=out_specs,
        out_shape=out_shape,
    )(r, sel, cbm2_l, cn_l)
    return idx, rout


_SC_CHUNK = 128


def _sc_gather(table, idx):
    info = plsc.get_sparse_core_info()
    nw = info.num_cores * info.num_subcores
    b_per_w = N // nw
    n_chunks = b_per_w // _SC_CHUNK
    mesh = plsc.VectorSubcoreMesh(core_axis_name="c", subcore_axis_name="s")

    @functools.partial(
        pl.kernel, mesh=mesh,
        out_type=jax.ShapeDtypeStruct((N, D), jnp.float32),
        scratch_types=[
            pltpu.VMEM((n_chunks, _SC_CHUNK), jnp.int32),
            pltpu.VMEM((_SC_CHUNK, D), jnp.float32),
            pltpu.SemaphoreType.DMA,
        ],
    )
    def k(table_hbm, idx_hbm, out_hbm, idx_v, rows_v, sem):
        wid = lax.axis_index("s") * info.num_cores + lax.axis_index("c")
        base = wid * b_per_w
        for c in range(n_chunks):
            pltpu.sync_copy(
                idx_hbm.at[pl.ds(base + c * _SC_CHUNK, _SC_CHUNK)],
                idx_v.at[c])
        for c in range(n_chunks):
            pltpu.async_copy(table_hbm.at[idx_v.at[c]], rows_v, sem).wait()
            pltpu.sync_copy(
                rows_v, out_hbm.at[pl.ds(base + c * _SC_CHUNK, _SC_CHUNK)])

    return k(table, idx)


@jax.jit
def kernel(x, codebooks):
    cnorms = jnp.sum(codebooks * codebooks, axis=-1)
    cbm2 = -2.0 * codebooks
    codes = []
    r = x
    sel = x
    for level in range(N_LEVELS):
        subtract = level > 0
        write_r = level < N_LEVELS - 1
        idx, rout = _tc_level(
            r, sel,
            lax.slice_in_dim(cbm2, level, level + 1, axis=0),
            lax.slice_in_dim(cnorms, level, level + 1, axis=0),
            subtract, write_r)
        codes.append(idx)
        if write_r:
            sel = _sc_gather(codebooks[level], idx)
            r = rout
    return jnp.stack(codes, axis=-1)


# speedup vs baseline: 1.0272x; 1.0272x over previous
import functools

import jax
import jax.numpy as jnp
from jax import lax
from jax.experimental import pallas as pl
from jax.experimental.pallas import tpu as pltpu
from jax.experimental.pallas import tpu_sc as plsc

N_LEVELS = 8
K = 1024
D = 256
N = 16384
BLOCK_B = 2048
N_SPLIT = 4


def _tc_level_kernel(r_ref, sel_ref, cbm2_ref, cnorm_ref, idx_ref, rout_ref,
                     *, subtract, write_r):
    b = r_ref.shape[0]
    h = b // N_SPLIT
    lane_iota = jax.lax.broadcasted_iota(jnp.int32, (h, K), 1)

    def chain(j):
        sl = pl.ds(j * h, h)
        r = r_ref[sl, :]
        if subtract:
            r = r - sel_ref[sl, :]
        if write_r:
            rout_ref[sl, :] = r
        m2p = jax.lax.dot_general(
            r, cbm2_ref[0], (((1,), (1,)), ((), ())),
            preferred_element_type=jnp.float32,
        )
        d2 = cnorm_ref[0][None, :] + m2p
        idx_ref[sl] = jnp.argmin(d2, axis=1).astype(jnp.int32)

    for j in range(N_SPLIT):
        chain(j)


def _tc_level(r, sel, cbm2_l, cn_l, subtract, write_r):
    grid = (N // BLOCK_B,)
    body = functools.partial(_tc_level_kernel, subtract=subtract,
                             write_r=write_r)
    in_specs = [
        pl.BlockSpec((BLOCK_B, D), lambda i: (i, 0)),
        pl.BlockSpec((BLOCK_B, D), lambda i: (i, 0)),
        pl.BlockSpec((1, K, D), lambda i: (0, 0, 0)),
        pl.BlockSpec((1, K), lambda i: (0, 0)),
    ]
    out_specs = [
        pl.BlockSpec((BLOCK_B,), lambda i: (i,)),
        pl.BlockSpec((BLOCK_B, D), lambda i: (i, 0)),
    ]
    out_shape = [
        jax.ShapeDtypeStruct((N,), jnp.int32),
        jax.ShapeDtypeStruct((N, D), jnp.float32),
    ]
    idx, rout = pl.pallas_call(
        body,
        grid=grid,
        in_specs=in_specs,
        out_specs=out_specs,
        out_shape=out_shape,
    )(r, sel, cbm2_l, cn_l)
    return idx, rout


_SC_CHUNK = 128


def _sc_gather(table, idx):
    info = plsc.get_sparse_core_info()
    nw = info.num_cores * info.num_subcores
    b_per_w = N // nw
    n_chunks = b_per_w // _SC_CHUNK
    mesh = plsc.VectorSubcoreMesh(core_axis_name="c", subcore_axis_name="s")

    @functools.partial(
        pl.kernel, mesh=mesh,
        out_type=jax.ShapeDtypeStruct((N, D), jnp.float32),
        scratch_types=[
            pltpu.VMEM((n_chunks, _SC_CHUNK), jnp.int32),
            pltpu.VMEM((_SC_CHUNK, D), jnp.float32),
            pltpu.VMEM((_SC_CHUNK, D), jnp.float32),
            pltpu.VMEM((_SC_CHUNK, D), jnp.float32),
            pltpu.SemaphoreType.DMA,
        ],
    )
    def k(table_hbm, idx_hbm, out_hbm, idx_v, r0, r1, r2, sem):
        wid = lax.axis_index("s") * info.num_cores + lax.axis_index("c")
        base = wid * b_per_w
        for c in range(n_chunks):
            pltpu.sync_copy(
                idx_hbm.at[pl.ds(base + c * _SC_CHUNK, _SC_CHUNK)],
                idx_v.at[c])
        bufs = (r0, r1, r2)
        cps = [pltpu.async_copy(table_hbm.at[idx_v.at[c]], bufs[c], sem)
               for c in range(3)]
        for cp in cps:
            cp.wait()
        pltpu.sync_copy(r0, out_hbm.at[pl.ds(base, _SC_CHUNK)])
        cp3 = pltpu.async_copy(table_hbm.at[idx_v.at[3]], r0, sem)
        pltpu.sync_copy(r1, out_hbm.at[pl.ds(base + _SC_CHUNK, _SC_CHUNK)])
        pltpu.sync_copy(r2, out_hbm.at[pl.ds(base + 2 * _SC_CHUNK, _SC_CHUNK)])
        cp3.wait()
        pltpu.sync_copy(r0, out_hbm.at[pl.ds(base + 3 * _SC_CHUNK, _SC_CHUNK)])

    return k(table, idx)


@jax.jit
def kernel(x, codebooks):
    cnorms = jnp.sum(codebooks * codebooks, axis=-1)
    cbm2 = -2.0 * codebooks
    codes = []
    r = x
    sel = x
    for level in range(N_LEVELS):
        subtract = level > 0
        write_r = level < N_LEVELS - 1
        idx, rout = _tc_level(
            r, sel,
            lax.slice_in_dim(cbm2, level, level + 1, axis=0),
            lax.slice_in_dim(cnorms, level, level + 1, axis=0),
            subtract, write_r)
        codes.append(idx)
        if write_r:
            sel = _sc_gather(codebooks[level], idx)
            r = rout
    return jnp.stack(codes, axis=-1)


# speedup vs baseline: 1.4266x; 1.3888x over previous
import jax
import jax.numpy as jnp
from jax.experimental import pallas as pl
from jax.experimental.pallas import tpu as pltpu

N_LEVELS = 8
K = 1024
D = 256
BLOCK_B = 2048
N_SPLIT = 2


def _rvq_kernel(x_ref, cbm2_ref, csplit_ref, cnorm_ref, out_ref):
    b = x_ref.shape[0]
    h = b // N_SPLIT
    lane_iota = jax.lax.broadcasted_iota(jnp.int32, (h, K), 1)

    def level_step(r, level, row0):
        m2p = jax.lax.dot_general(
            r, cbm2_ref[level], (((1,), (1,)), ((), ())),
            preferred_element_type=jnp.float32,
        )
        d2 = cnorm_ref[level][None, :] + m2p
        idx = jnp.argmin(d2, axis=1).astype(jnp.int32)
        out_ref[pl.ds(row0, h), level] = idx
        if level == N_LEVELS - 1:
            return r
        onehot = (lane_iota == idx[:, None]).astype(jnp.float32).astype(jnp.bfloat16)
        s = jax.lax.dot_general(
            onehot, csplit_ref[level], (((1,), (0,)), ((), ())),
            preferred_element_type=jnp.float32,
        )
        sel = (s[:, :D] + s[:, D:2 * D]) + s[:, 2 * D:]
        return r - sel

    rs = [x_ref[pl.ds(j * h, h), :] for j in range(N_SPLIT)]
    for level in range(N_LEVELS):
        rs = [level_step(rs[j], level, j * h) for j in range(N_SPLIT)]


@jax.jit
def kernel(x, codebooks):
    n = x.shape[0]
    cnorms = jnp.sum(codebooks * codebooks, axis=-1)
    cbm2 = -2.0 * codebooks
    mask = jnp.uint32(0xFFFF0000)
    bits = jax.lax.bitcast_convert_type(codebooks, jnp.uint32)
    b0 = jax.lax.bitcast_convert_type(bits & mask, jnp.float32)
    r1 = codebooks - b0
    b1 = jax.lax.bitcast_convert_type(
        jax.lax.bitcast_convert_type(r1, jnp.uint32) & mask, jnp.float32)
    b2 = r1 - b1
    csplit = jnp.concatenate(
        [b0.astype(jnp.bfloat16), b1.astype(jnp.bfloat16),
         b2.astype(jnp.bfloat16)], axis=-1)

    grid = (n // BLOCK_B,)
    out = pl.pallas_call(
        _rvq_kernel,
        grid=grid,
        in_specs=[
            pl.BlockSpec((BLOCK_B, D), lambda i: (i, 0)),
            pl.BlockSpec((N_LEVELS, K, D), lambda i: (0, 0, 0)),
            pl.BlockSpec((N_LEVELS, K, 3 * D), lambda i: (0, 0, 0)),
            pl.BlockSpec((N_LEVELS, K), lambda i: (0, 0)),
        ],
        out_specs=pl.BlockSpec((BLOCK_B, N_LEVELS), lambda i: (i, 0)),
        out_shape=jax.ShapeDtypeStruct((n, N_LEVELS), jnp.int32),
        compiler_params=pltpu.CompilerParams(
            dimension_semantics=("parallel",)),
    )(x, cbm2, csplit, cnorms)
    return out
